# baseline (speedup 1.0000x reference)
import functools

import jax
import jax.numpy as jnp
from jax import lax
from jax.experimental import pallas as pl
from jax.experimental.pallas import tpu as pltpu
from jax.experimental.pallas import tpu_sc as plsc

NC = 2
NS = 16
NW = NC * NS
B = 112
LANES = 16


def _column_perm(d):
    import numpy as np
    pidx = np.concatenate(
        [np.concatenate([np.arange(g, g + 32, 2), np.arange(g + 1, g + 32, 2)])
         for g in range(0, d, 32)])
    return pidx, np.argsort(pidx)


@functools.lru_cache(maxsize=None)
def _build_deg_kernel(n, pt):
    mesh = plsc.VectorSubcoreMesh(core_axis_name="c", subcore_axis_name="s",
                                  num_cores=NC, num_subcores=NS)

    @functools.partial(
        pl.kernel,
        out_type=jax.ShapeDtypeStruct((NW, 1, n), jnp.float32),
        mesh=mesh,
        scratch_types=[
            pltpu.VMEM((pt,), jnp.int32),
            pltpu.VMEM((pt,), jnp.float32),
            pltpu.VMEM((n,), jnp.float32),
        ],
        compiler_params=pltpu.CompilerParams(needs_layout_passes=False),
    )
    def deg_kernel(dst_hbm, ew_hbm, out_hbm, dst_v, ew_v, deg_v):
        cid = lax.axis_index("c")
        sid = lax.axis_index("s")
        wid = cid * NS + sid

        pltpu.sync_copy(dst_hbm.at[pl.ds(wid * pt, pt)], dst_v)
        pltpu.sync_copy(ew_hbm.at[pl.ds(wid * pt, pt)], ew_v)

        def zero_body(i, _):
            deg_v[pl.ds(i * LANES, LANES)] = jnp.zeros((LANES,), jnp.float32)
            return _
        lax.fori_loop(0, n // LANES, zero_body, None)

        def scat_body(i, _):
            idx = dst_v[pl.ds(i * LANES, LANES)]
            w = ew_v[pl.ds(i * LANES, LANES)]
            plsc.addupdate_scatter(deg_v, [idx], w)
            return _
        lax.fori_loop(0, pt // LANES, scat_body, None)

        pltpu.sync_copy(deg_v, out_hbm.at[wid, 0])

    return deg_kernel


@functools.lru_cache(maxsize=None)
def _build_spmm_kernel(n_pad, d, nb0, nb1):
    mesh = plsc.VectorSubcoreMesh(core_axis_name="c", subcore_axis_name="s",
                                  num_cores=NC, num_subcores=NS)
    rows_per_tile = n_pad // NS
    n_full = rows_per_tile // B
    n_rem = rows_per_tile - n_full * B
    dh = d // 2

    assert nb0 % 4 == 0 and nb1 % 4 == 0 and 4 <= nb1 <= nb0
    assert rows_per_tile % 8 == 0

    @functools.partial(
        pl.kernel,
        out_type=jax.ShapeDtypeStruct((NC, n_pad, d), jnp.float32),
        mesh=mesh,
        scratch_types=[
            pltpu.VMEM((3, B), jnp.int32),
            pltpu.VMEM((3, B), jnp.int32),
            pltpu.VMEM((3, B), jnp.int32),
            pltpu.VMEM((3, B), jnp.int32),
            pltpu.VMEM((B, dh), jnp.int32),
            pltpu.VMEM((B, dh), jnp.int32),
            pltpu.VMEM((B, d), jnp.float32),
            pltpu.VMEM((B, d), jnp.float32),
            pltpu.VMEM_SHARED((n_pad, d), jnp.float32),
            pltpu.SemaphoreType.DMA,
            pltpu.SemaphoreType.DMA,
            pltpu.SemaphoreType.DMA,
            pltpu.SemaphoreType.DMA,
            pltpu.SemaphoreType.DMA,
            pltpu.SemaphoreType.DMA,
            pltpu.SemaphoreType.DMA,
            pltpu.SemaphoreType.DMA,
        ],
        compiler_params=pltpu.CompilerParams(needs_layout_passes=False,
                                             use_tc_tiling_on_sc=False),
    )
    def spmm_kernel(se_hbm, xsp_hbm, out_hbm,
                    se0, se1, se2, se3, gb0, gb1, rows0, rows1, acc_sh,
                    e0, e1, e2, e3, g0, g1, s0, s1):
        cid = lax.axis_index("c")
        sid = lax.axis_index("s")
        wid = cid * NS + sid
        my_nb = jnp.where(cid == 0, nb0, nb1)
        se = (se0, se1, se2, se3)
        gb = (gb0, gb1)
        rows = (rows0, rows1)
        esem = (e0, e1, e2, e3)
        gsem = (g0, g1)
        ssem = (s0, s1)

        def zrow(e, _):
            for r in rows:
                for c in range(d // LANES):
                    r[e, pl.ds(c * LANES, LANES)] = (
                        jnp.zeros((LANES,), jnp.float32))
            return _
        lax.fori_loop(0, B, zrow, None)
        base = sid * rows_per_tile
        for k in range(n_full):
            pltpu.sync_copy(rows0, acc_sh.at[pl.ds(base + k * B, B)])
        if n_rem:
            pltpu.sync_copy(rows0.at[pl.ds(0, n_rem)],
                            acc_sh.at[pl.ds(base + n_full * B, n_rem)])
        plsc.subcore_barrier()

        pltpu.async_copy(se_hbm.at[wid, 0], se0, esem[0])
        pltpu.async_copy(se_hbm.at[wid, 1], se1, esem[1])
        pltpu.make_async_copy(se_hbm.at[wid, 0], se0, esem[0]).wait()
        for k in range(2):
            pltpu.async_copy(rows[k], acc_sh.at[se0.at[1]], ssem[k],
                             add=True)
        pltpu.async_copy(xsp_hbm.at[se0.at[0]], gb0, gsem[0])

        def scale_rows(r, g_v, s_v):
            def scale(g, _):
                ewv = plsc.bitcast(s_v[2, pl.ds(g * LANES, LANES)],
                                   jnp.float32)
                for l in range(LANES):
                    s = ewv[l]
                    e = g * LANES + l
                    for c in range(d // (2 * LANES)):
                        pw = g_v[e, pl.ds(c * LANES, LANES)]
                        a = plsc.bitcast(pw << 16, jnp.float32)
                        bv = plsc.bitcast(pw & jnp.int32(-65536),
                                          jnp.float32)
                        r[e, pl.ds(c * 2 * LANES, LANES)] = a * s
                        r[e, pl.ds((c * 2 + 1) * LANES, LANES)] = bv * s
                return _
            lax.fori_loop(0, B // LANES, scale, None)

        def batch_body(jj, _):
            for k in range(4):
                j = jj * 4 + k
                b2 = k % 2
                pltpu.make_async_copy(
                    xsp_hbm.at[se[k].at[0]], gb[b2], gsem[b2]).wait()

                @pl.when(j < my_nb - 1)
                def _():
                    pltpu.make_async_copy(
                        se_hbm.at[wid, 0], se[(k + 1) % 4],
                        esem[(k + 1) % 4]).wait()
                    pltpu.async_copy(
                        xsp_hbm.at[se[(k + 1) % 4].at[0]], gb[1 - b2],
                        gsem[1 - b2])

                pltpu.make_async_copy(
                    rows[b2], acc_sh.at[se[k].at[1]], ssem[b2]).wait()

                scale_rows(rows[b2], gb[b2], se[k])

                @pl.when(j < my_nb - 2)
                def _():
                    pltpu.async_copy(se_hbm.at[wid, j + 2], se[(k + 2) % 4],
                                     esem[(k + 2) % 4])

                pltpu.async_copy(rows[b2], acc_sh.at[se[k].at[1]], ssem[b2],
                                 add=True)
            return _
        lax.fori_loop(0, my_nb // 4, batch_body, None)

        for k in range(2):
            pltpu.make_async_copy(
                rows[k], acc_sh.at[se0.at[1]], ssem[k]).wait()

        plsc.subcore_barrier()
        pltpu.sync_copy(acc_sh.at[pl.ds(base, rows_per_tile)],
                        out_hbm.at[cid, pl.ds(base, rows_per_tile)])

    return spmm_kernel


def _dis_body(degp_ref, dis_ref):
    deg = jnp.sum(degp_ref[...], axis=0) + 1.0
    dis = jnp.where(deg > 0, lax.rsqrt(deg), 0.0)
    dis_ref[...] = dis[:, None]


@functools.lru_cache(maxsize=None)
def _build_dis(n):
    return pl.pallas_call(
        _dis_body,
        out_shape=jax.ShapeDtypeStruct((n, 1), jnp.float32),
    )


def _xs_body(x_ref, dis_ref, w_ref, xs_ref):
    xs_ref[...] = dis_ref[...] * jnp.dot(
        x_ref[...], w_ref[...], preferred_element_type=jnp.float32)


@functools.lru_cache(maxsize=None)
def _build_xs(n, d, bn):
    grid = n // bn
    return pl.pallas_call(
        _xs_body,
        grid=(grid,),
        in_specs=[
            pl.BlockSpec((bn, d), lambda i: (i, 0)),
            pl.BlockSpec((bn, 1), lambda i: (i, 0)),
            pl.BlockSpec((d, d), lambda i: (0, 0)),
        ],
        out_specs=pl.BlockSpec((bn, d), lambda i: (i, 0)),
        out_shape=jax.ShapeDtypeStruct((n, d), jnp.float32),
    )


def _combine_body(acc_ref, xs_ref, dis_ref, b_ref, w_ref, out_ref):
    t = acc_ref[0] + acc_ref[1] + xs_ref[...]
    pre = dis_ref[...] * t + b_ref[...]
    h = pre * jax.nn.sigmoid(pre)
    out_ref[...] = dis_ref[...] * jnp.dot(
        h, w_ref[...], preferred_element_type=jnp.float32)


@functools.lru_cache(maxsize=None)
def _build_combine(n, d, bn):
    grid = n // bn
    return pl.pallas_call(
        _combine_body,
        grid=(grid,),
        in_specs=[
            pl.BlockSpec((NC, bn, d), lambda i: (0, i, 0)),
            pl.BlockSpec((bn, d), lambda i: (i, 0)),
            pl.BlockSpec((bn, 1), lambda i: (i, 0)),
            pl.BlockSpec((1, d), lambda i: (0, 0)),
            pl.BlockSpec((d, d), lambda i: (0, 0)),
        ],
        out_specs=pl.BlockSpec((bn, d), lambda i: (i, 0)),
        out_shape=jax.ShapeDtypeStruct((n, d), jnp.float32),
    )


@functools.lru_cache(maxsize=None)
def _build_final(n, d, bn):
    grid = n // bn
    inv_n = 1.0 / n

    def body(acc_ref, xs_ref, dis_ref, b_ref, out_ref):
        i = pl.program_id(0)
        t = acc_ref[0] + acc_ref[1] + xs_ref[...]
        pre = dis_ref[...] * t + b_ref[...]
        h = pre * jax.nn.sigmoid(pre)
        part = jnp.sum(h, axis=0, keepdims=True) * inv_n

        @pl.when(i == 0)
        def _():
            out_ref[...] = part

        @pl.when(i > 0)
        def _():
            out_ref[...] += part

    return pl.pallas_call(
        body,
        grid=(grid,),
        in_specs=[
            pl.BlockSpec((NC, bn, d), lambda i: (0, i, 0)),
            pl.BlockSpec((bn, d), lambda i: (i, 0)),
            pl.BlockSpec((bn, 1), lambda i: (i, 0)),
            pl.BlockSpec((1, d), lambda i: (0, 0)),
        ],
        out_specs=pl.BlockSpec((1, d), lambda i: (0, 0)),
        out_shape=jax.ShapeDtypeStruct((1, d), jnp.float32),
    )


def kernel(x, edge_index, edge_weight, W1, b1, W2, b2, W3, b3):
    n, d = x.shape
    e = edge_weight.shape[0]
    src, dst = edge_index[0], edge_index[1]

    frac0 = 0.67
    nbt = (e + B - 1) // B
    nb0 = max(4, int(round(frac0 * nbt / NS / 4)) * 4)
    nb1 = max(4, -((NS * nb0 - nbt) // NS) // 4 * 4 + 4)
    while NS * (nb0 + nb1) * B < e:
        nb1 += 4
    ep = NS * (nb0 + nb1) * B
    pad = ep - e
    pt = ep // NW
    srcp = jnp.concatenate([src, jnp.zeros((pad,), src.dtype)])
    dstp = jnp.concatenate([dst, jnp.zeros((pad,), dst.dtype)])
    ewp = jnp.concatenate([edge_weight, jnp.zeros((pad,), edge_weight.dtype)])
    ewbits = lax.bitcast_convert_type(ewp, jnp.int32)

    def tile_layout(a):
        cut = NS * nb0 * B
        p0 = a[:cut].reshape(NS, nb0, B)
        p1 = a[cut:].reshape(NS, nb1, B)
        p1 = jnp.concatenate(
            [p1, jnp.zeros((NS, nb0 - nb1, B), a.dtype)], axis=1)
        return jnp.concatenate([p0, p1], axis=0)

    se2 = jnp.stack([tile_layout(srcp), tile_layout(dstp),
                     tile_layout(ewbits)], axis=2)

    bn = 2000
    n_pad = ((n + NS * 8 - 1) // (NS * 8)) * (NS * 8)
    pidx, pinv = _column_perm(d)
    deg_part = _build_deg_kernel(n, pt)(dstp, ewp)
    dis = _build_dis(n)(deg_part.reshape(NW, n))
    xs = _build_xs(n, d, bn)(x, dis, W1[:, pidx])

    def pack_rows(xss):
        xb = xss.astype(jnp.bfloat16).reshape(n, d // 32, 2, 16)
        pairs = jnp.stack([xb[:, :, 0, :], xb[:, :, 1, :]], axis=-1)
        return lax.bitcast_convert_type(pairs, jnp.int32).reshape(n, d // 2)

    spmm = _build_spmm_kernel(n_pad, d, nb0, nb1)
    combine = _build_combine(n, d, bn)

    b1r = b1[pidx].reshape(1, d)
    b2r = b2[pidx].reshape(1, d)
    b3r = b3[pidx].reshape(1, d)
    W2p = W2[pidx][:, pidx]
    W3p = W3[pidx][:, pidx]

    acc = spmm(se2, pack_rows(xs))
    xs = combine(acc, xs, dis, b1r, W2p)
    acc = spmm(se2, pack_rows(xs))
    xs = combine(acc, xs, dis, b2r, W3p)
    acc = spmm(se2, pack_rows(xs))
    out = _build_final(n, d, bn)(acc, xs, dis, b3r)
    return out[:, pinv]


# speedup vs baseline: 1.4845x; 1.4845x over previous
import functools

import jax
import jax.numpy as jnp
from jax import lax
from jax.experimental import pallas as pl
from jax.experimental.pallas import tpu as pltpu
from jax.experimental.pallas import tpu_sc as plsc

NC = 2
NS = 16
NW = NC * NS
B = 128
LANES = 16


def _column_perm(d):
    import numpy as np
    pidx = np.concatenate(
        [np.concatenate([np.arange(g, g + 32, 2), np.arange(g + 1, g + 32, 2)])
         for g in range(0, d, 32)])
    return pidx, np.argsort(pidx)


@functools.lru_cache(maxsize=None)
def _build_deg_kernel(n, pt):
    mesh = plsc.VectorSubcoreMesh(core_axis_name="c", subcore_axis_name="s",
                                  num_cores=NC, num_subcores=NS)

    @functools.partial(
        pl.kernel,
        out_type=jax.ShapeDtypeStruct((NW, 1, n), jnp.float32),
        mesh=mesh,
        scratch_types=[
            pltpu.VMEM((pt,), jnp.int32),
            pltpu.VMEM((pt,), jnp.float32),
            pltpu.VMEM((n,), jnp.float32),
        ],
        compiler_params=pltpu.CompilerParams(needs_layout_passes=False),
    )
    def deg_kernel(dst_hbm, ew_hbm, out_hbm, dst_v, ew_v, deg_v):
        cid = lax.axis_index("c")
        sid = lax.axis_index("s")
        wid = cid * NS + sid

        pltpu.sync_copy(dst_hbm.at[pl.ds(wid * pt, pt)], dst_v)
        pltpu.sync_copy(ew_hbm.at[pl.ds(wid * pt, pt)], ew_v)

        def zero_body(i, _):
            deg_v[pl.ds(i * LANES, LANES)] = jnp.zeros((LANES,), jnp.float32)
            return _
        lax.fori_loop(0, n // LANES, zero_body, None)

        def scat_body(i, _):
            idx = dst_v[pl.ds(i * LANES, LANES)]
            w = ew_v[pl.ds(i * LANES, LANES)]
            plsc.addupdate_scatter(deg_v, [idx], w)
            return _
        lax.fori_loop(0, pt // LANES, scat_body, None)

        pltpu.sync_copy(deg_v, out_hbm.at[wid, 0])

    return deg_kernel


@functools.lru_cache(maxsize=None)
def _build_spmm_kernel(n_pad, d, nb0, nb1):
    mesh = plsc.VectorSubcoreMesh(core_axis_name="c", subcore_axis_name="s",
                                  num_cores=NC, num_subcores=NS)
    rows_per_tile = n_pad // NS
    n_full = rows_per_tile // B
    n_rem = rows_per_tile - n_full * B

    assert nb0 % 2 == 0 and nb1 % 2 == 0 and 2 <= nb1 <= nb0
    assert rows_per_tile % 8 == 0

    @functools.partial(
        pl.kernel,
        out_type=jax.ShapeDtypeStruct((NC, n_pad, d), jnp.float32),
        mesh=mesh,
        scratch_types=[
            pltpu.VMEM((nb0, B), jnp.int32),
            pltpu.VMEM((2, B), jnp.int32),
            pltpu.VMEM((2, B), jnp.int32),
            pltpu.VMEM((B, d), jnp.float32),
            pltpu.VMEM((B, d), jnp.float32),
            pltpu.VMEM_SHARED((n_pad, d), jnp.float32),
            pltpu.SemaphoreType.DMA,
            pltpu.SemaphoreType.DMA,
            pltpu.SemaphoreType.DMA,
            pltpu.SemaphoreType.DMA,
            pltpu.SemaphoreType.DMA,
            pltpu.SemaphoreType.DMA,
        ],
        compiler_params=pltpu.CompilerParams(needs_layout_passes=False),
    )
    def spmm_kernel(dst_hbm, se_hbm, xs_hbm, out_hbm,
                    dst_v, se0, se1, rows0, rows1, acc_sh,
                    e0, e1, g0, g1, s0, s1):
        cid = lax.axis_index("c")
        sid = lax.axis_index("s")
        wid = cid * NS + sid
        my_nb = jnp.where(cid == 0, nb0, nb1)
        se = (se0, se1)
        rows = (rows0, rows1)
        esem = (e0, e1)
        gsem = (g0, g1)
        ssem = (s0, s1)

        pltpu.sync_copy(dst_hbm.at[wid], dst_v)

        def zrow(e, _):
            for r in rows:
                for c in range(d // LANES):
                    r[e, pl.ds(c * LANES, LANES)] = (
                        jnp.zeros((LANES,), jnp.float32))
            return _
        lax.fori_loop(0, B, zrow, None)
        base = sid * rows_per_tile
        for k in range(n_full):
            pltpu.sync_copy(rows0, acc_sh.at[pl.ds(base + k * B, B)])
        if n_rem:
            pltpu.sync_copy(rows0.at[pl.ds(0, n_rem)],
                            acc_sh.at[pl.ds(base + n_full * B, n_rem)])
        plsc.subcore_barrier()

        pltpu.async_copy(se_hbm.at[wid, 0], se0, esem[0])
        pltpu.async_copy(se_hbm.at[wid, 1], se1, esem[1])
        for k in range(2):
            pltpu.async_copy(rows[k], acc_sh.at[dst_v.at[0]], ssem[k],
                             add=True)
        pltpu.make_async_copy(rows0, acc_sh.at[dst_v.at[0]], ssem[0]).wait()
        pltpu.make_async_copy(se_hbm.at[wid, 0], se0, esem[0]).wait()
        pltpu.async_copy(xs_hbm.at[se0.at[0]], rows0, gsem[0])

        def scale_rows(r, s_v):
            def scale(g, _):
                ewv = plsc.bitcast(s_v[1, pl.ds(g * LANES, LANES)],
                                   jnp.float32)
                for l in range(LANES):
                    s = ewv[l]
                    e = g * LANES + l
                    for c in range(d // LANES):
                        sl = pl.ds(c * LANES, LANES)
                        r[e, sl] = r[e, sl] * s
                return _
            lax.fori_loop(0, B // LANES, scale, None)

        def batch_body(jj, _):
            for k in range(2):
                j = jj * 2 + k
                b, b1 = k, 1 - k
                pltpu.make_async_copy(
                    xs_hbm.at[se[b].at[0]], rows[b], gsem[b]).wait()

                @pl.when(j < my_nb - 1)
                def _():
                    pltpu.make_async_copy(
                        se_hbm.at[wid, 0], se[b1], esem[b1]).wait()
                    pltpu.make_async_copy(
                        rows[b1], acc_sh.at[dst_v.at[0]], ssem[b1]).wait()
                    pltpu.async_copy(
                        xs_hbm.at[se[b1].at[0]], rows[b1], gsem[b1])

                scale_rows(rows[b], se[b])

                @pl.when(j < my_nb - 2)
                def _():
                    pltpu.async_copy(se_hbm.at[wid, j + 2], se[b], esem[b])

                pltpu.async_copy(rows[b], acc_sh.at[dst_v.at[j]], ssem[b],
                                 add=True)
            return _
        lax.fori_loop(0, my_nb // 2, batch_body, None)

        for k in range(2):
            pltpu.make_async_copy(
                rows[k], acc_sh.at[dst_v.at[0]], ssem[k]).wait()

        plsc.subcore_barrier()
        pltpu.sync_copy(acc_sh.at[pl.ds(base, rows_per_tile)],
                        out_hbm.at[cid, pl.ds(base, rows_per_tile)])

    return spmm_kernel


def _dis_body(degp_ref, dis_ref):
    deg = jnp.sum(degp_ref[...], axis=0) + 1.0
    dis = jnp.where(deg > 0, lax.rsqrt(deg), 0.0)
    dis_ref[...] = dis[:, None]


@functools.lru_cache(maxsize=None)
def _build_dis(n):
    return pl.pallas_call(
        _dis_body,
        out_shape=jax.ShapeDtypeStruct((n, 1), jnp.float32),
    )


def _xs_body(x_ref, dis_ref, w_ref, xs_ref):
    xs_ref[...] = dis_ref[...] * jnp.dot(
        x_ref[...], w_ref[...], preferred_element_type=jnp.float32)


@functools.lru_cache(maxsize=None)
def _build_xs(n, d, bn):
    grid = n // bn
    return pl.pallas_call(
        _xs_body,
        grid=(grid,),
        in_specs=[
            pl.BlockSpec((bn, d), lambda i: (i, 0)),
            pl.BlockSpec((bn, 1), lambda i: (i, 0)),
            pl.BlockSpec((d, d), lambda i: (0, 0)),
        ],
        out_specs=pl.BlockSpec((bn, d), lambda i: (i, 0)),
        out_shape=jax.ShapeDtypeStruct((n, d), jnp.float32),
    )


def _combine_body(acc_ref, xs_ref, dis_ref, b_ref, w_ref, out_ref):
    t = acc_ref[0] + acc_ref[1] + xs_ref[...]
    pre = dis_ref[...] * t + b_ref[...]
    h = pre * jax.nn.sigmoid(pre)
    out_ref[...] = dis_ref[...] * jnp.dot(
        h, w_ref[...], preferred_element_type=jnp.float32)


@functools.lru_cache(maxsize=None)
def _build_combine(n, d, bn):
    grid = n // bn
    return pl.pallas_call(
        _combine_body,
        grid=(grid,),
        in_specs=[
            pl.BlockSpec((NC, bn, d), lambda i: (0, i, 0)),
            pl.BlockSpec((bn, d), lambda i: (i, 0)),
            pl.BlockSpec((bn, 1), lambda i: (i, 0)),
            pl.BlockSpec((1, d), lambda i: (0, 0)),
            pl.BlockSpec((d, d), lambda i: (0, 0)),
        ],
        out_specs=pl.BlockSpec((bn, d), lambda i: (i, 0)),
        out_shape=jax.ShapeDtypeStruct((n, d), jnp.float32),
    )


@functools.lru_cache(maxsize=None)
def _build_final(n, d, bn):
    grid = n // bn
    inv_n = 1.0 / n

    def body(acc_ref, xs_ref, dis_ref, b_ref, out_ref):
        i = pl.program_id(0)
        t = acc_ref[0] + acc_ref[1] + xs_ref[...]
        pre = dis_ref[...] * t + b_ref[...]
        h = pre * jax.nn.sigmoid(pre)
        part = jnp.sum(h, axis=0, keepdims=True) * inv_n

        @pl.when(i == 0)
        def _():
            out_ref[...] = part

        @pl.when(i > 0)
        def _():
            out_ref[...] += part

    return pl.pallas_call(
        body,
        grid=(grid,),
        in_specs=[
            pl.BlockSpec((NC, bn, d), lambda i: (0, i, 0)),
            pl.BlockSpec((bn, d), lambda i: (i, 0)),
            pl.BlockSpec((bn, 1), lambda i: (i, 0)),
            pl.BlockSpec((1, d), lambda i: (0, 0)),
        ],
        out_specs=pl.BlockSpec((1, d), lambda i: (0, 0)),
        out_shape=jax.ShapeDtypeStruct((1, d), jnp.float32),
    )


def kernel(x, edge_index, edge_weight, W1, b1, W2, b2, W3, b3):
    n, d = x.shape
    e = edge_weight.shape[0]
    src, dst = edge_index[0], edge_index[1]

    frac0 = 0.76
    nbt = (e + B - 1) // B
    nb0 = max(2, int(round(frac0 * nbt / NS / 2)) * 2)
    nb1 = max(2, -((NS * nb0 - nbt) // NS) // 2 * 2 + 2)
    while NS * (nb0 + nb1) * B < e:
        nb1 += 2
    ep = NS * (nb0 + nb1) * B
    pad = ep - e
    pt = ep // NW
    srcp = jnp.concatenate([src, jnp.zeros((pad,), src.dtype)])
    dstp = jnp.concatenate([dst, jnp.zeros((pad,), dst.dtype)])
    ewp = jnp.concatenate([edge_weight, jnp.zeros((pad,), edge_weight.dtype)])
    ewbits = lax.bitcast_convert_type(ewp, jnp.int32)

    def tile_layout(a):
        cut = NS * nb0 * B
        p0 = a[:cut].reshape(NS, nb0, B)
        p1 = a[cut:].reshape(NS, nb1, B)
        p1 = jnp.concatenate(
            [p1, jnp.zeros((NS, nb0 - nb1, B), a.dtype)], axis=1)
        return jnp.concatenate([p0, p1], axis=0)

    dst2 = tile_layout(dstp)
    se2 = jnp.stack([tile_layout(srcp), tile_layout(ewbits)],
                    axis=2)

    bn = 2000
    n_pad = ((n + NS * 8 - 1) // (NS * 8)) * (NS * 8)
    deg_part = _build_deg_kernel(n, pt)(dstp, ewp)
    dis = _build_dis(n)(deg_part.reshape(NW, n))
    xs = _build_xs(n, d, bn)(x, dis, W1)

    spmm = _build_spmm_kernel(n_pad, d, nb0, nb1)
    combine = _build_combine(n, d, bn)

    b1r = b1.reshape(1, d)
    b2r = b2.reshape(1, d)
    b3r = b3.reshape(1, d)

    acc = spmm(dst2, se2, xs)
    xs = combine(acc, xs, dis, b1r, W2)
    acc = spmm(dst2, se2, xs)
    xs = combine(acc, xs, dis, b2r, W3)
    acc = spmm(dst2, se2, xs)
    out = _build_final(n, d, bn)(acc, xs, dis, b3r)
    return out


# speedup vs baseline: 1.5720x; 1.0589x over previous
import functools

import jax
import jax.numpy as jnp
from jax import lax
from jax.experimental import pallas as pl
from jax.experimental.pallas import tpu as pltpu
from jax.experimental.pallas import tpu_sc as plsc

NC = 2
NS = 16
NW = NC * NS
B = 128
LANES = 16


def _column_perm(d):
    import numpy as np
    pidx = np.concatenate(
        [np.concatenate([np.arange(g, g + 32, 2), np.arange(g + 1, g + 32, 2)])
         for g in range(0, d, 32)])
    return pidx, np.argsort(pidx)


@functools.lru_cache(maxsize=None)
def _build_deg_kernel(n, pt):
    mesh = plsc.VectorSubcoreMesh(core_axis_name="c", subcore_axis_name="s",
                                  num_cores=NC, num_subcores=NS)

    @functools.partial(
        pl.kernel,
        out_type=jax.ShapeDtypeStruct((NW, 1, n), jnp.float32),
        mesh=mesh,
        scratch_types=[
            pltpu.VMEM((pt,), jnp.int32),
            pltpu.VMEM((pt,), jnp.float32),
            pltpu.VMEM((n,), jnp.float32),
        ],
        compiler_params=pltpu.CompilerParams(needs_layout_passes=False),
    )
    def deg_kernel(dst_hbm, ew_hbm, out_hbm, dst_v, ew_v, deg_v):
        cid = lax.axis_index("c")
        sid = lax.axis_index("s")
        wid = cid * NS + sid

        pltpu.sync_copy(dst_hbm.at[pl.ds(wid * pt, pt)], dst_v)
        pltpu.sync_copy(ew_hbm.at[pl.ds(wid * pt, pt)], ew_v)

        def zero_body(i, _):
            deg_v[pl.ds(i * LANES, LANES)] = jnp.zeros((LANES,), jnp.float32)
            return _
        lax.fori_loop(0, n // LANES, zero_body, None)

        def scat_body(i, _):
            idx = dst_v[pl.ds(i * LANES, LANES)]
            w = ew_v[pl.ds(i * LANES, LANES)]
            plsc.addupdate_scatter(deg_v, [idx], w)
            return _
        lax.fori_loop(0, pt // LANES, scat_body, None)

        pltpu.sync_copy(deg_v, out_hbm.at[wid, 0])

    return deg_kernel


@functools.lru_cache(maxsize=None)
def _build_spmm_kernel(n_pad, d, nb0, nb1):
    mesh = plsc.VectorSubcoreMesh(core_axis_name="c", subcore_axis_name="s",
                                  num_cores=NC, num_subcores=NS)
    rows_per_tile = n_pad // NS
    n_full = rows_per_tile // B
    n_rem = rows_per_tile - n_full * B

    assert nb0 % 2 == 0 and nb1 % 2 == 0 and 2 <= nb1 <= nb0
    assert rows_per_tile % 8 == 0

    @functools.partial(
        pl.kernel,
        out_type=jax.ShapeDtypeStruct((NC, n_pad, d), jnp.float32),
        mesh=mesh,
        scratch_types=[
            pltpu.VMEM((nb0, B), jnp.int32),
            pltpu.VMEM((2, B), jnp.int32),
            pltpu.VMEM((2, B), jnp.int32),
            pltpu.VMEM((B, d), jnp.float32),
            pltpu.VMEM((B, d), jnp.float32),
            pltpu.VMEM_SHARED((n_pad, d), jnp.float32),
            pltpu.SemaphoreType.DMA,
            pltpu.SemaphoreType.DMA,
            pltpu.SemaphoreType.DMA,
            pltpu.SemaphoreType.DMA,
            pltpu.SemaphoreType.DMA,
            pltpu.SemaphoreType.DMA,
        ],
        compiler_params=pltpu.CompilerParams(needs_layout_passes=False),
    )
    def spmm_kernel(dst_hbm, se_hbm, xs_hbm, out_hbm,
                    dst_v, se0, se1, rows0, rows1, acc_sh,
                    e0, e1, g0, g1, s0, s1):
        cid = lax.axis_index("c")
        sid = lax.axis_index("s")
        wid = cid * NS + sid
        my_nb = jnp.where(cid == 0, nb0, nb1)
        se = (se0, se1)
        rows = (rows0, rows1)
        esem = (e0, e1)
        gsem = (g0, g1)
        ssem = (s0, s1)

        pltpu.sync_copy(dst_hbm.at[wid], dst_v)

        def zrow(e, _):
            for r in rows:
                for c in range(d // LANES):
                    r[e, pl.ds(c * LANES, LANES)] = (
                        jnp.zeros((LANES,), jnp.float32))
            return _
        lax.fori_loop(0, B, zrow, None)
        base = sid * rows_per_tile
        for k in range(n_full):
            pltpu.sync_copy(rows0, acc_sh.at[pl.ds(base + k * B, B)])
        if n_rem:
            pltpu.sync_copy(rows0.at[pl.ds(0, n_rem)],
                            acc_sh.at[pl.ds(base + n_full * B, n_rem)])
        plsc.subcore_barrier()

        pltpu.async_copy(se_hbm.at[wid, 0], se0, esem[0])
        pltpu.async_copy(se_hbm.at[wid, 1], se1, esem[1])
        for k in range(2):
            pltpu.async_copy(rows[k], acc_sh.at[dst_v.at[0]], ssem[k],
                             add=True)
        pltpu.make_async_copy(rows0, acc_sh.at[dst_v.at[0]], ssem[0]).wait()
        pltpu.make_async_copy(se_hbm.at[wid, 0], se0, esem[0]).wait()
        pltpu.async_copy(xs_hbm.at[se0.at[0]], rows0, gsem[0])

        def scale_rows(r, s_v):
            def scale(g, _):
                ewv = plsc.bitcast(s_v[1, pl.ds(g * LANES, LANES)],
                                   jnp.float32)
                for l in range(LANES):
                    s = ewv[l]
                    e = g * LANES + l
                    for c in range(d // LANES):
                        sl = pl.ds(c * LANES, LANES)
                        r[e, sl] = r[e, sl] * s
                return _
            lax.fori_loop(0, B // LANES, scale, None)

        def batch_body(jj, _):
            for k in range(2):
                j = jj * 2 + k
                b, b1 = k, 1 - k
                pltpu.make_async_copy(
                    xs_hbm.at[se[b].at[0]], rows[b], gsem[b]).wait()

                @pl.when(j < my_nb - 1)
                def _():
                    pltpu.make_async_copy(
                        se_hbm.at[wid, 0], se[b1], esem[b1]).wait()
                    pltpu.make_async_copy(
                        rows[b1], acc_sh.at[dst_v.at[0]], ssem[b1]).wait()
                    pltpu.async_copy(
                        xs_hbm.at[se[b1].at[0]], rows[b1], gsem[b1])

                scale_rows(rows[b], se[b])

                @pl.when(j < my_nb - 2)
                def _():
                    pltpu.async_copy(se_hbm.at[wid, j + 2], se[b], esem[b])

                pltpu.async_copy(rows[b], acc_sh.at[dst_v.at[j]], ssem[b],
                                 add=True)
            return _
        lax.fori_loop(0, my_nb // 2, batch_body, None)

        for k in range(2):
            pltpu.make_async_copy(
                rows[k], acc_sh.at[dst_v.at[0]], ssem[k]).wait()

        plsc.subcore_barrier()
        pltpu.sync_copy(acc_sh.at[pl.ds(base, rows_per_tile)],
                        out_hbm.at[cid, pl.ds(base, rows_per_tile)])

    return spmm_kernel


def _dis_body(degp_ref, dis_ref):
    deg = jnp.sum(degp_ref[...], axis=0) + 1.0
    dis = jnp.where(deg > 0, lax.rsqrt(deg), 0.0)
    dis_ref[...] = dis[:, None]


@functools.lru_cache(maxsize=None)
def _build_dis(n):
    return pl.pallas_call(
        _dis_body,
        out_shape=jax.ShapeDtypeStruct((n, 1), jnp.float32),
    )


def _xs_body(x_ref, dis_ref, w_ref, xs_ref):
    xs_ref[...] = dis_ref[...] * jnp.dot(
        x_ref[...], w_ref[...], preferred_element_type=jnp.float32)


@functools.lru_cache(maxsize=None)
def _build_xs(n, d, bn):
    grid = n // bn
    return pl.pallas_call(
        _xs_body,
        grid=(grid,),
        in_specs=[
            pl.BlockSpec((bn, d), lambda i: (i, 0)),
            pl.BlockSpec((bn, 1), lambda i: (i, 0)),
            pl.BlockSpec((d, d), lambda i: (0, 0)),
        ],
        out_specs=pl.BlockSpec((bn, d), lambda i: (i, 0)),
        out_shape=jax.ShapeDtypeStruct((n, d), jnp.float32),
    )


def _combine_body(acc_ref, xs_ref, dis_ref, b_ref, w_ref, out_ref):
    t = acc_ref[0] + acc_ref[1] + xs_ref[...]
    pre = dis_ref[...] * t + b_ref[...]
    h = pre * jax.nn.sigmoid(pre)
    out_ref[...] = dis_ref[...] * jnp.dot(
        h, w_ref[...], preferred_element_type=jnp.float32)


@functools.lru_cache(maxsize=None)
def _build_combine(n, d, bn):
    grid = n // bn
    return pl.pallas_call(
        _combine_body,
        grid=(grid,),
        in_specs=[
            pl.BlockSpec((NC, bn, d), lambda i: (0, i, 0)),
            pl.BlockSpec((bn, d), lambda i: (i, 0)),
            pl.BlockSpec((bn, 1), lambda i: (i, 0)),
            pl.BlockSpec((1, d), lambda i: (0, 0)),
            pl.BlockSpec((d, d), lambda i: (0, 0)),
        ],
        out_specs=pl.BlockSpec((bn, d), lambda i: (i, 0)),
        out_shape=jax.ShapeDtypeStruct((n, d), jnp.float32),
    )


@functools.lru_cache(maxsize=None)
def _build_final(n, d, bn):
    grid = n // bn
    inv_n = 1.0 / n

    def body(acc_ref, xs_ref, dis_ref, b_ref, out_ref):
        i = pl.program_id(0)
        t = acc_ref[0] + acc_ref[1] + xs_ref[...]
        pre = dis_ref[...] * t + b_ref[...]
        h = pre * jax.nn.sigmoid(pre)
        part = jnp.sum(h, axis=0, keepdims=True) * inv_n

        @pl.when(i == 0)
        def _():
            out_ref[...] = part

        @pl.when(i > 0)
        def _():
            out_ref[...] += part

    return pl.pallas_call(
        body,
        grid=(grid,),
        in_specs=[
            pl.BlockSpec((NC, bn, d), lambda i: (0, i, 0)),
            pl.BlockSpec((bn, d), lambda i: (i, 0)),
            pl.BlockSpec((bn, 1), lambda i: (i, 0)),
            pl.BlockSpec((1, d), lambda i: (0, 0)),
        ],
        out_specs=pl.BlockSpec((1, d), lambda i: (0, 0)),
        out_shape=jax.ShapeDtypeStruct((1, d), jnp.float32),
    )


def kernel(x, edge_index, edge_weight, W1, b1, W2, b2, W3, b3):
    n, d = x.shape
    e = edge_weight.shape[0]
    src, dst = edge_index[0], edge_index[1]

    frac0 = 0.78
    nbt = (e + B - 1) // B
    nb0 = max(2, int(round(frac0 * nbt / NS / 2)) * 2)
    nb1 = max(2, -((NS * nb0 - nbt) // NS) // 2 * 2 + 2)
    while NS * (nb0 + nb1) * B < e:
        nb1 += 2
    ep = NS * (nb0 + nb1) * B
    pad = ep - e
    pt = ep // NW
    srcp = jnp.concatenate([src, jnp.zeros((pad,), src.dtype)])
    dstp = jnp.concatenate([dst, jnp.zeros((pad,), dst.dtype)])
    ewp = jnp.concatenate([edge_weight, jnp.zeros((pad,), edge_weight.dtype)])
    ewbits = lax.bitcast_convert_type(ewp, jnp.int32)

    def tile_layout(a):
        cut = NS * nb0 * B
        p0 = a[:cut].reshape(NS, nb0, B)
        p1 = a[cut:].reshape(NS, nb1, B)
        p1 = jnp.concatenate(
            [p1, jnp.zeros((NS, nb0 - nb1, B), a.dtype)], axis=1)
        return jnp.concatenate([p0, p1], axis=0)

    dst2 = tile_layout(dstp)
    se2 = jnp.stack([tile_layout(srcp), tile_layout(ewbits)],
                    axis=2)

    bn = 2000
    n_pad = ((n + NS * 8 - 1) // (NS * 8)) * (NS * 8)
    deg_part = _build_deg_kernel(n, pt)(dstp, ewp)
    dis = _build_dis(n)(deg_part.reshape(NW, n))
    xs = _build_xs(n, d, bn)(x, dis, W1)

    spmm = _build_spmm_kernel(n_pad, d, nb0, nb1)
    combine = _build_combine(n, d, bn)

    b1r = b1.reshape(1, d)
    b2r = b2.reshape(1, d)
    b3r = b3.reshape(1, d)

    acc = spmm(dst2, se2, xs)
    xs = combine(acc, xs, dis, b1r, W2)
    acc = spmm(dst2, se2, xs)
    xs = combine(acc, xs, dis, b2r, W3)
    acc = spmm(dst2, se2, xs)
    out = _build_final(n, d, bn)(acc, xs, dis, b3r)
    return out
